# baseline (speedup 1.0000x reference)
import jax
import jax.numpy as jnp
from jax.experimental import pallas as pl
from jax.experimental.pallas import tpu as pltpu


def _pos_kernel(row_ref, col_ref, out_hbm, tile_ref, sem):
    h, d = row_ref.shape
    w = col_ref.shape[0]
    b = out_hbm.shape[0]
    tile_ref[:, :, 0:d] = jnp.broadcast_to(col_ref[...][None, :, :], (h, w, d))
    tile_ref[:, :, d : 2 * d] = jnp.broadcast_to(row_ref[...][:, None, :], (h, w, d))
    split = 2
    hs = h // split
    copies = [
        pltpu.make_async_copy(
            tile_ref.at[pl.ds(s * hs, hs)],
            out_hbm.at[i, pl.ds(s * hs, hs)],
            sem.at[i * split + s],
        )
        for i in range(b)
        for s in range(split)
    ]
    for c in copies:
        c.start()
    for c in copies:
        c.wait()


def kernel(x, row_embed, col_embed):
    b = x.shape[0]
    h, w = x.shape[-3], x.shape[-2]
    d = row_embed.shape[-1]
    out = pl.pallas_call(
        _pos_kernel,
        in_specs=[
            pl.BlockSpec(memory_space=pltpu.MemorySpace.VMEM),
            pl.BlockSpec(memory_space=pltpu.MemorySpace.VMEM),
        ],
        out_specs=pl.BlockSpec(memory_space=pltpu.MemorySpace.HBM),
        out_shape=jax.ShapeDtypeStruct((b, h, w, 2 * d), row_embed.dtype),
        scratch_shapes=[
            pltpu.VMEM((h, w, 2 * d), row_embed.dtype),
            pltpu.SemaphoreType.DMA((2 * b,)),
        ],
    )(row_embed, col_embed)
    return out.reshape(b, h * w, 2 * d)
